# baseline (speedup 1.0000x reference)
import functools

import jax
import jax.numpy as jnp
from jax import lax
from jax.experimental import pallas as pl
from jax.experimental.pallas import tpu as pltpu
from jax.experimental.pallas import tpu_sc as plsc

NC = 2
NS = 16
NW = NC * NS
L = 16
CHUNK = 112
GROUPS = CHUNK // L
LAMBDA_SEMANTIC = 0.01


def _rsqrt(x):
    i = plsc.bitcast(x, jnp.int32)
    y = plsc.bitcast(jnp.int32(0x5F3759DF) - (i >> 1), jnp.float32)
    half = x * 0.5
    for _ in range(4):
        y = y * (1.5 - half * y * y)
    return y


def _make_sc_kernel(V, D, nchunk):
    mesh = plsc.VectorSubcoreMesh(
        core_axis_name="c", subcore_axis_name="s", num_cores=NC, num_subcores=NS
    )

    @functools.partial(
        pl.kernel,
        out_type=jax.ShapeDtypeStruct((2 * NW, L), jnp.float32),
        mesh=mesh,
        compiler_params=pltpu.CompilerParams(use_tc_tiling_on_sc=False, needs_layout_passes=False),
        scratch_types=[
            pltpu.VMEM((nchunk, CHUNK), jnp.int32),
            pltpu.VMEM((nchunk, CHUNK), jnp.int32),
            pltpu.VMEM((nchunk, CHUNK), jnp.float32),
            pltpu.VMEM((2, CHUNK, D), jnp.float32),
            pltpu.VMEM((2, CHUNK, D), jnp.float32),
            pltpu.VMEM((2, L), jnp.float32),
            pltpu.VMEM((2, L), jnp.int32),
            pltpu.SemaphoreType.DMA,
            pltpu.SemaphoreType.DMA,
            pltpu.SemaphoreType.DMA,
            pltpu.SemaphoreType.DMA,
        ],
    )
    def sc_kernel(cb, i1, i2, sc, out, i1v, i2v, scv, r1v, r2v, accv, lanev,
                  sem1a, sem2a, sem1b, sem2b):
        c = lax.axis_index("c")
        s = lax.axis_index("s")
        wid = s * NC + c

        rowbase = wid * nchunk
        pltpu.sync_copy(i1.at[pl.ds(rowbase, nchunk)], i1v)
        pltpu.sync_copy(i2.at[pl.ds(rowbase, nchunk)], i2v)
        pltpu.sync_copy(sc.at[pl.ds(rowbase, nchunk)], scv)

        lane = lax.iota(jnp.int32, L)
        zero = jnp.zeros((L,), jnp.float32)
        one = zero + 1.0
        lanev.at[0][...] = lane
        lanev.at[1][...] = lane

        def start(j, buf, s1, s2):
            pltpu.async_copy(cb.at[i1v.at[j]], r1v.at[buf], s1)
            pltpu.async_copy(cb.at[i2v.at[j]], r2v.at[buf], s2)

        def wait(j, buf, s1, s2):
            pltpu.make_async_copy(cb.at[i1v.at[j]], r1v.at[buf], s1).wait()
            pltpu.make_async_copy(cb.at[i2v.at[j]], r2v.at[buf], s2).wait()

        def compute(j, buf, lane_r, carry):
            acc_s, acc_n = carry
            r1b = r1v.at[buf]
            r2b = r2v.at[buf]
            for g in range(GROUPS):
                base = g * L
                rowidx = base + lane_r

                @plsc.parallel_loop(0, D, unroll=4, carry=(zero, zero, zero))
                def dstep(d, c3):
                    dot, n1, n2 = c3
                    col = (lane_r + d) & (D - 1)
                    v1 = plsc.load_gather(r1b, [rowidx, col])
                    v2 = plsc.load_gather(r2b, [rowidx, col])
                    return dot + v1 * v2, n1 + v1 * v1, n2 + v2 * v2

                dot, n1, n2 = dstep

                i1g = i1v.at[j][pl.ds(base, L)]
                i2g = i2v.at[j][pl.ds(base, L)]
                s_g = scv.at[j][pl.ds(base, L)]
                valid = (i1g != i2g) & (i1g < V) & (i2g < V)
                vf = jnp.where(valid, one, zero)
                dsq = jnp.maximum(n1, 1e-24) * jnp.maximum(n2, 1e-24)
                cos = dot * _rsqrt(dsq)
                acc_s = acc_s + cos * s_g * vf
                acc_n = acc_n + vf
            return acc_s, acc_n

        nhalf = nchunk // 2
        start(0, 0, sem1a, sem2a)

        def body2(jj, carry):
            j0 = 2 * jj
            lane_r = lanev.at[jj & 1][...]
            wait(j0, 0, sem1a, sem2a)
            start(j0 + 1, 1, sem1b, sem2b)
            carry = compute(j0, 0, lane_r, carry)
            wait(j0 + 1, 1, sem1b, sem2b)

            @pl.when(jj + 1 < nhalf)
            def _():
                start(j0 + 2, 0, sem1a, sem2a)

            return compute(j0 + 1, 1, lane_r, carry)

        acc_s, acc_n = lax.fori_loop(0, nhalf, body2, (zero, zero))
        accv.at[0][...] = acc_s
        accv.at[1][...] = acc_n
        pltpu.sync_copy(accv, out.at[pl.ds(wid * 2, 2)])

    return sc_kernel


def kernel(codebook, pair_idx1, pair_idx2, pair_scores):
    V, D = codebook.shape
    P = pair_idx1.shape[0]
    per_super = NW * CHUNK
    nchunk = -(-P // per_super)
    nchunk += nchunk % 2
    p_pad = per_super * nchunk
    pad = p_pad - P
    if pad:
        zi = (jnp.arange(pad, dtype=pair_idx1.dtype) * 8) % V
        pair_idx1 = jnp.concatenate([pair_idx1, zi])
        pair_idx2 = jnp.concatenate([pair_idx2, zi])
        pair_scores = jnp.concatenate([pair_scores, jnp.zeros((pad,), pair_scores.dtype)])
    i1r = pair_idx1.reshape(NW * nchunk, CHUNK)
    i2r = pair_idx2.reshape(NW * nchunk, CHUNK)
    scr = pair_scores.reshape(NW * nchunk, CHUNK)

    out = _make_sc_kernel(V, D, nchunk)(codebook, i1r, i2r, scr)
    total = jnp.sum(out[0::2])
    n_valid = jnp.sum(out[1::2])
    avg = jnp.where(n_valid > 0, total / jnp.maximum(n_valid, 1.0), 0.0)
    return jnp.where(n_valid > 0, -LAMBDA_SEMANTIC * avg, 0.0)


# speedup vs baseline: 1.0396x; 1.0396x over previous
import functools

import jax
import jax.numpy as jnp
from jax import lax
from jax.experimental import pallas as pl
from jax.experimental.pallas import tpu as pltpu
from jax.experimental.pallas import tpu_sc as plsc

NC = 2
NS = 16
NW = NC * NS
L = 16
CHUNK = 112
GROUPS = CHUNK // L
LAMBDA_SEMANTIC = 0.01


def _rsqrt(x):
    i = plsc.bitcast(x, jnp.int32)
    y = plsc.bitcast(jnp.int32(0x5F3759DF) - (i >> 1), jnp.float32)
    half = x * 0.5
    for _ in range(4):
        y = y * (1.5 - half * y * y)
    return y


def _make_sc_kernel(V, D, nchunk):
    mesh = plsc.VectorSubcoreMesh(
        core_axis_name="c", subcore_axis_name="s", num_cores=NC, num_subcores=NS
    )

    @functools.partial(
        pl.kernel,
        out_type=jax.ShapeDtypeStruct((2 * NW, L), jnp.float32),
        mesh=mesh,
        compiler_params=pltpu.CompilerParams(use_tc_tiling_on_sc=False, needs_layout_passes=False),
        scratch_types=[
            pltpu.VMEM((nchunk, CHUNK), jnp.int32),
            pltpu.VMEM((nchunk, CHUNK), jnp.int32),
            pltpu.VMEM((nchunk, CHUNK), jnp.float32),
            pltpu.VMEM((3, CHUNK, D), jnp.float32),
            pltpu.VMEM((3, CHUNK, D), jnp.float32),
            pltpu.VMEM((2, L), jnp.float32),
            pltpu.VMEM((3, L), jnp.int32),
            [pltpu.SemaphoreType.DMA] * 3,
            [pltpu.SemaphoreType.DMA] * 3,
        ],
    )
    def sc_kernel(cb, i1, i2, sc, out, i1v, i2v, scv, r1v, r2v, accv, lanev,
                  sem1, sem2):
        c = lax.axis_index("c")
        s = lax.axis_index("s")
        wid = s * NC + c

        rowbase = wid * nchunk
        pltpu.sync_copy(i1.at[pl.ds(rowbase, nchunk)], i1v)
        pltpu.sync_copy(i2.at[pl.ds(rowbase, nchunk)], i2v)
        pltpu.sync_copy(sc.at[pl.ds(rowbase, nchunk)], scv)

        lane = lax.iota(jnp.int32, L)
        zero = jnp.zeros((L,), jnp.float32)
        one = zero + 1.0
        lanev.at[0][...] = lane
        lanev.at[1][...] = lane
        lanev.at[2][...] = lane

        def start(j, buf):
            pltpu.async_copy(cb.at[i1v.at[j]], r1v.at[buf], sem1[buf])
            pltpu.async_copy(cb.at[i2v.at[j]], r2v.at[buf], sem2[buf])

        def wait(j, buf):
            pltpu.make_async_copy(cb.at[i1v.at[j]], r1v.at[buf], sem1[buf]).wait()
            pltpu.make_async_copy(cb.at[i2v.at[j]], r2v.at[buf], sem2[buf]).wait()

        def compute(j, buf, lane_r, carry):
            acc_s, acc_n = carry
            r1b = r1v.at[buf]
            r2b = r2v.at[buf]
            for g in range(GROUPS):
                base = g * L
                rowidx = base + lane_r

                @plsc.parallel_loop(0, D, unroll=4, carry=(zero, zero, zero))
                def dstep(d, c3):
                    dot, n1, n2 = c3
                    col = (lane_r + d) & (D - 1)
                    v1 = plsc.load_gather(r1b, [rowidx, col])
                    v2 = plsc.load_gather(r2b, [rowidx, col])
                    return dot + v1 * v2, n1 + v1 * v1, n2 + v2 * v2

                dot, n1, n2 = dstep

                i1g = i1v.at[j][pl.ds(base, L)]
                i2g = i2v.at[j][pl.ds(base, L)]
                s_g = scv.at[j][pl.ds(base, L)]
                valid = (i1g != i2g) & (i1g < V) & (i2g < V)
                vf = jnp.where(valid, one, zero)
                dsq = jnp.maximum(n1, 1e-24) * jnp.maximum(n2, 1e-24)
                cos = dot * _rsqrt(dsq)
                acc_s = acc_s + cos * s_g * vf
                acc_n = acc_n + vf
            return acc_s, acc_n

        ntrip = nchunk // 3
        start(0, 0)
        start(1, 1)

        def body3(jj, carry):
            j0 = 3 * jj
            lane_r = lanev.at[jj % 3][...]
            for b in range(3):
                j = j0 + b
                wait(j, b)

                @pl.when(j + 2 < nchunk)
                def _():
                    start(j + 2, (b + 2) % 3)

                carry = compute(j, b, lane_r, carry)
            return carry

        acc_s, acc_n = lax.fori_loop(0, ntrip, body3, (zero, zero))
        accv.at[0][...] = acc_s
        accv.at[1][...] = acc_n
        pltpu.sync_copy(accv, out.at[pl.ds(wid * 2, 2)])

    return sc_kernel


def kernel(codebook, pair_idx1, pair_idx2, pair_scores):
    V, D = codebook.shape
    P = pair_idx1.shape[0]
    per_super = NW * CHUNK
    nchunk = -(-P // per_super)
    nchunk += -nchunk % 3
    p_pad = per_super * nchunk
    pad = p_pad - P
    if pad:
        zi = (jnp.arange(pad, dtype=pair_idx1.dtype) * 8) % V
        pair_idx1 = jnp.concatenate([pair_idx1, zi])
        pair_idx2 = jnp.concatenate([pair_idx2, zi])
        pair_scores = jnp.concatenate([pair_scores, jnp.zeros((pad,), pair_scores.dtype)])
    i1r = pair_idx1.reshape(NW * nchunk, CHUNK)
    i2r = pair_idx2.reshape(NW * nchunk, CHUNK)
    scr = pair_scores.reshape(NW * nchunk, CHUNK)

    out = _make_sc_kernel(V, D, nchunk)(codebook, i1r, i2r, scr)
    total = jnp.sum(out[0::2])
    n_valid = jnp.sum(out[1::2])
    avg = jnp.where(n_valid > 0, total / jnp.maximum(n_valid, 1.0), 0.0)
    return jnp.where(n_valid > 0, -LAMBDA_SEMANTIC * avg, 0.0)


# speedup vs baseline: 1.0446x; 1.0048x over previous
import functools

import jax
import jax.numpy as jnp
from jax import lax
from jax.experimental import pallas as pl
from jax.experimental.pallas import tpu as pltpu
from jax.experimental.pallas import tpu_sc as plsc

NC = 2
NS = 16
NW = NC * NS
L = 16
CHUNK = 112
GROUPS = CHUNK // L
LAMBDA_SEMANTIC = 0.01


def _rsqrt(x):
    i = plsc.bitcast(x, jnp.int32)
    y = plsc.bitcast(jnp.int32(0x5F3759DF) - (i >> 1), jnp.float32)
    half = x * 0.5
    for _ in range(4):
        y = y * (1.5 - half * y * y)
    return y


def _make_sc_kernel(V, D, nchunk):
    mesh = plsc.VectorSubcoreMesh(
        core_axis_name="c", subcore_axis_name="s", num_cores=NC, num_subcores=NS
    )

    @functools.partial(
        pl.kernel,
        out_type=jax.ShapeDtypeStruct((2 * NW, L), jnp.float32),
        mesh=mesh,
        compiler_params=pltpu.CompilerParams(use_tc_tiling_on_sc=False, needs_layout_passes=False),
        scratch_types=[
            pltpu.VMEM((nchunk, CHUNK), jnp.int32),
            pltpu.VMEM((nchunk, CHUNK), jnp.int32),
            pltpu.VMEM((nchunk, CHUNK), jnp.float32),
            pltpu.VMEM((4, CHUNK, D), jnp.float32),
            pltpu.VMEM((4, CHUNK, D), jnp.float32),
            pltpu.VMEM((2, L), jnp.float32),
            pltpu.VMEM((4, L), jnp.int32),
            [pltpu.SemaphoreType.DMA] * 4,
            [pltpu.SemaphoreType.DMA] * 4,
        ],
    )
    def sc_kernel(cb, i1, i2, sc, out, i1v, i2v, scv, r1v, r2v, accv, lanev,
                  sem1, sem2):
        c = lax.axis_index("c")
        s = lax.axis_index("s")
        wid = s * NC + c

        rowbase = wid * nchunk
        pltpu.sync_copy(i1.at[pl.ds(rowbase, nchunk)], i1v)
        pltpu.sync_copy(i2.at[pl.ds(rowbase, nchunk)], i2v)
        pltpu.sync_copy(sc.at[pl.ds(rowbase, nchunk)], scv)

        lane = lax.iota(jnp.int32, L)
        zero = jnp.zeros((L,), jnp.float32)
        one = zero + 1.0
        lanev.at[0][...] = lane
        lanev.at[1][...] = lane
        lanev.at[2][...] = lane
        lanev.at[3][...] = lane

        def start(j, buf):
            pltpu.async_copy(cb.at[i1v.at[j]], r1v.at[buf], sem1[buf])
            pltpu.async_copy(cb.at[i2v.at[j]], r2v.at[buf], sem2[buf])

        def wait(j, buf):
            pltpu.make_async_copy(cb.at[i1v.at[j]], r1v.at[buf], sem1[buf]).wait()
            pltpu.make_async_copy(cb.at[i2v.at[j]], r2v.at[buf], sem2[buf]).wait()

        def compute(j, buf, lane_r, carry):
            acc_s, acc_n = carry
            r1b = r1v.at[buf]
            r2b = r2v.at[buf]
            for g in range(GROUPS):
                base = g * L
                rowidx = base + lane_r

                @plsc.parallel_loop(0, D, unroll=4, carry=(zero, zero, zero))
                def dstep(d, c3):
                    dot, n1, n2 = c3
                    col = (lane_r + d) & (D - 1)
                    v1 = plsc.load_gather(r1b, [rowidx, col])
                    v2 = plsc.load_gather(r2b, [rowidx, col])
                    return dot + v1 * v2, n1 + v1 * v1, n2 + v2 * v2

                dot, n1, n2 = dstep

                i1g = i1v.at[j][pl.ds(base, L)]
                i2g = i2v.at[j][pl.ds(base, L)]
                s_g = scv.at[j][pl.ds(base, L)]
                valid = (i1g != i2g) & (i1g < V) & (i2g < V)
                vf = jnp.where(valid, one, zero)
                dsq = jnp.maximum(n1, 1e-24) * jnp.maximum(n2, 1e-24)
                cos = dot * _rsqrt(dsq)
                acc_s = acc_s + cos * s_g * vf
                acc_n = acc_n + vf
            return acc_s, acc_n

        ntrip = nchunk // 4
        start(0, 0)
        start(1, 1)
        start(2, 2)

        def body4(jj, carry):
            j0 = 4 * jj
            lane_r = lanev.at[jj & 3][...]
            for b in range(4):
                j = j0 + b
                wait(j, b)

                @pl.when(j + 3 < nchunk)
                def _():
                    start(j + 3, (b + 3) % 4)

                carry = compute(j, b, lane_r, carry)
            return carry

        acc_s, acc_n = lax.fori_loop(0, ntrip, body4, (zero, zero))
        accv.at[0][...] = acc_s
        accv.at[1][...] = acc_n
        pltpu.sync_copy(accv, out.at[pl.ds(wid * 2, 2)])

    return sc_kernel


def kernel(codebook, pair_idx1, pair_idx2, pair_scores):
    V, D = codebook.shape
    P = pair_idx1.shape[0]
    per_super = NW * CHUNK
    nchunk = -(-P // per_super)
    nchunk += -nchunk % 4
    p_pad = per_super * nchunk
    pad = p_pad - P
    if pad:
        zi = (jnp.arange(pad, dtype=pair_idx1.dtype) * 8) % V
        pair_idx1 = jnp.concatenate([pair_idx1, zi])
        pair_idx2 = jnp.concatenate([pair_idx2, zi])
        pair_scores = jnp.concatenate([pair_scores, jnp.zeros((pad,), pair_scores.dtype)])
    i1r = pair_idx1.reshape(NW * nchunk, CHUNK)
    i2r = pair_idx2.reshape(NW * nchunk, CHUNK)
    scr = pair_scores.reshape(NW * nchunk, CHUNK)

    out = _make_sc_kernel(V, D, nchunk)(codebook, i1r, i2r, scr)
    total = jnp.sum(out[0::2])
    n_valid = jnp.sum(out[1::2])
    avg = jnp.where(n_valid > 0, total / jnp.maximum(n_valid, 1.0), 0.0)
    return jnp.where(n_valid > 0, -LAMBDA_SEMANTIC * avg, 0.0)
